# baseline (speedup 1.0000x reference)
import functools

import jax
import jax.numpy as jnp
from jax import lax
from jax.experimental import pallas as pl
from jax.experimental.pallas import tpu as pltpu
from jax.experimental.pallas import tpu_sc as plsc

_NW = 32
_R = 32
_NBUF = 2


def _sc_body(s, b, f, table_hbm, out_hbm, bufs, in_sems, out_sems):
    wid = lax.axis_index("s") * 2 + lax.axis_index("c")
    rows_per_w = s // _NW
    nch = rows_per_w // _R
    base = wid * rows_per_w

    def in_copy(i):
        return pltpu.make_async_copy(
            table_hbm.at[pl.ds(base + i * _R, _R), :],
            bufs.at[i % _NBUF],
            in_sems.at[i % _NBUF],
        )

    def out_copy(i, j):
        return pltpu.make_async_copy(
            bufs.at[i % _NBUF],
            out_hbm.at[pl.ds(base + i * _R, _R), pl.ds(j * f, f)],
            out_sems.at[i % _NBUF],
        )

    for i in range(min(_NBUF, nch)):
        in_copy(i).start()
    for i in range(nch):
        if i >= _NBUF:
            for j in range(b):
                out_copy(i - _NBUF, j).wait()
            in_copy(i).start()
        in_copy(i).wait()
        for j in range(b):
            out_copy(i, j).start()
    for i in range(max(0, nch - _NBUF), nch):
        for j in range(b):
            out_copy(i, j).wait()


def kernel(x, table):
    s, b, f = x.shape
    mesh = plsc.VectorSubcoreMesh(core_axis_name="c", subcore_axis_name="s")
    sc_call = functools.partial(
        pl.kernel,
        mesh=mesh,
        out_type=jax.ShapeDtypeStruct((s, b * f), jnp.float32),
        scratch_types=[
            pltpu.VMEM((_NBUF, _R, f), jnp.float32),
            pltpu.SemaphoreType.DMA((_NBUF,)),
            pltpu.SemaphoreType.DMA((_NBUF,)),
        ],
    )

    @sc_call
    def run(table_hbm, out_hbm, bufs, in_sems, out_sems):
        _sc_body(s, b, f, table_hbm, out_hbm, bufs, in_sems, out_sems)

    out2d = run(table)
    return out2d.reshape(s, b, f)


# speedup vs baseline: 3.8040x; 3.8040x over previous
import jax
import jax.numpy as jnp
from jax.experimental import pallas as pl

_S_BLK = 256


def _bcast_body(table_ref, out_ref):
    out_ref[...] = jnp.broadcast_to(table_ref[...][:, None, :], out_ref.shape)


def kernel(x, table):
    s, b, f = x.shape
    return pl.pallas_call(
        _bcast_body,
        grid=(s // _S_BLK,),
        in_specs=[pl.BlockSpec((_S_BLK, f), lambda i: (i, 0))],
        out_specs=pl.BlockSpec((_S_BLK, b, f), lambda i: (i, 0, 0)),
        out_shape=jax.ShapeDtypeStruct((s, b, f), table.dtype),
    )(table)


# speedup vs baseline: 4.3020x; 1.1309x over previous
import jax
import jax.numpy as jnp
from jax.experimental import pallas as pl

_S_BLK = 512


def _bcast_body(table_ref, out_ref):
    out_ref[...] = jnp.broadcast_to(table_ref[...][:, None, :], out_ref.shape)


def kernel(x, table):
    s, b, f = x.shape
    return pl.pallas_call(
        _bcast_body,
        grid=(s // _S_BLK,),
        in_specs=[pl.BlockSpec((_S_BLK, f), lambda i: (i, 0))],
        out_specs=pl.BlockSpec((_S_BLK, b, f), lambda i: (i, 0, 0)),
        out_shape=jax.ShapeDtypeStruct((s, b, f), table.dtype),
    )(table)


# speedup vs baseline: 4.5124x; 1.0489x over previous
import jax
import jax.numpy as jnp
from jax.experimental import pallas as pl
from jax.experimental.pallas import tpu as pltpu

_S_BLK = 512
_NBUF = 4


def _dma_body(s, b, f, table_hbm, out_hbm, bufs, in_sems, out_sems):
    n = s // _S_BLK

    def in_copy(i):
        return pltpu.make_async_copy(
            table_hbm.at[pl.ds(i * _S_BLK, _S_BLK), :],
            bufs.at[i % _NBUF],
            in_sems.at[i % _NBUF],
        )

    def out_copy(i, j):
        return pltpu.make_async_copy(
            bufs.at[i % _NBUF],
            out_hbm.at[pl.ds(i * _S_BLK, _S_BLK), j, :],
            out_sems.at[i % _NBUF],
        )

    for i in range(min(_NBUF, n)):
        in_copy(i).start()
    for i in range(n):
        if i >= _NBUF:
            for j in range(b):
                out_copy(i - _NBUF, j).wait()
            in_copy(i).start()
        in_copy(i).wait()
        for j in range(b):
            out_copy(i, j).start()
    for i in range(max(0, n - _NBUF), n):
        for j in range(b):
            out_copy(i, j).wait()


def kernel(x, table):
    s, b, f = x.shape
    return pl.pallas_call(
        lambda t, o, bufs, isem, osem: _dma_body(s, b, f, t, o, bufs, isem, osem),
        in_specs=[pl.BlockSpec(memory_space=pltpu.MemorySpace.HBM)],
        out_specs=pl.BlockSpec(memory_space=pltpu.MemorySpace.HBM),
        out_shape=jax.ShapeDtypeStruct((s, b, f), table.dtype),
        scratch_shapes=[
            pltpu.VMEM((_NBUF, _S_BLK, f), jnp.float32),
            pltpu.SemaphoreType.DMA((_NBUF,)),
            pltpu.SemaphoreType.DMA((_NBUF,)),
        ],
    )(table)
